# baseline (speedup 1.0000x reference)
import numpy as np
import jax
import jax.numpy as jnp
from jax.experimental import pallas as pl
from jax.experimental.pallas import tpu as pltpu


def _bilinear_matrix(out_size, in_size):
    scale = in_size / out_size
    idx = np.arange(out_size)
    src = (idx + 0.5) * scale - 0.5
    src = np.clip(src, 0.0, None)
    i0 = np.minimum(np.floor(src).astype(np.int64), in_size - 1)
    i1 = np.minimum(i0 + 1, in_size - 1)
    w1 = (src - i0).astype(np.float32)
    w0 = (1.0 - w1).astype(np.float32)
    A = np.zeros((out_size, in_size), dtype=np.float32)
    A[idx, i0] += w0
    A[idx, i1] += w1
    return A


def _fused_kernel(z_ref, wa_ref, wb_ref, bb_ref, wc_ref, bc_ref, ah_ref,
                  awt2_ref, out_ref):
    _, NCLS, H, W = out_ref.shape
    Hp, K18, W2o = z_ref.shape[1:]
    Ho = Hp - 1
    OC = wa_ref.shape[0]

    Z = z_ref[0]

    wa_b = jnp.broadcast_to(wa_ref[...], (Ho, OC, K18))
    wb_b = jnp.broadcast_to(wb_ref[...], (Ho, OC, K18))
    feat = jax.lax.dot_general(
        wa_b, Z[0:Ho], (((2,), (1,)), ((0,), (0,))),
        preferred_element_type=jnp.float32)
    feat = feat + jax.lax.dot_general(
        wb_b, Z[1:Ho + 1], (((2,), (1,)), ((0,), (0,))),
        preferred_element_type=jnp.float32)
    feat = jnp.maximum(feat + bb_ref[...][None], 0.0).astype(jnp.bfloat16)

    wc_b = jnp.broadcast_to(wc_ref[...], (Ho, NCLS, OC))
    logits = jax.lax.dot_general(
        wc_b, feat, (((2,), (1,)), ((0,), (0,))),
        preferred_element_type=jnp.float32)
    logits = (logits + bc_ref[...][None]).astype(jnp.bfloat16)

    awt_b = jnp.broadcast_to(awt2_ref[...], (Ho, W2o, 2 * W))
    t = jax.lax.dot_general(
        logits, awt_b, (((2,), (1,)), ((0,), (0,))),
        preferred_element_type=jnp.float32)
    t = t.astype(jnp.bfloat16)

    ah = ah_ref[...]
    for n in range(NCLS):
        y = jax.lax.dot_general(
            ah, t[:, n, :], (((1,), (0,)), ((), ())),
            preferred_element_type=jnp.float32)
        out_ref[0, n] = y[:, :W]
        out_ref[1, n] = y[:, W:]


def kernel(x, backbone_w, backbone_b, cls_w, cls_b):
    N, C, H, W = x.shape
    OC = backbone_w.shape[0]
    NCLS = cls_w.shape[0]
    Ho = (H + 2 - 3) // 2 + 1
    Wo = (W + 2 - 3) // 2 + 1
    Hp, Wp = Ho + 1, Wo + 1
    K18 = 6 * C

    xb = x.astype(jnp.bfloat16)
    xpad = jnp.pad(xb, ((0, 0), (0, 0), (1, 2 * Hp - H - 1),
                        (1, 2 * Wp - W - 1)))
    G = xpad.reshape(N // 2, 2, C, Hp, 2, Wp, 2)
    G = G.transpose(0, 3, 4, 2, 1, 6, 5)

    cv0 = G[:, :, :, :, :, 0, 0:Wo]
    cv1 = G[:, :, :, :, :, 1, 0:Wo]
    cv2 = G[:, :, :, :, :, 0, 1:Wo + 1]
    Z = jnp.stack([cv0, cv1, cv2], axis=3)
    Z = Z.reshape(N // 2, Hp, K18, 2 * Wo)

    wa = backbone_w[:, :, 0:2, :].transpose(0, 2, 3, 1).reshape(OC, K18)
    wb2 = jnp.concatenate(
        [backbone_w[:, :, 2, :].transpose(0, 2, 1).reshape(OC, 3 * C),
         jnp.zeros((OC, K18 - 3 * C), backbone_w.dtype)], axis=1)
    wa = wa.astype(jnp.bfloat16)
    wb2 = wb2.astype(jnp.bfloat16)

    bb2 = backbone_b.reshape(OC, 1).astype(jnp.float32)
    wc2 = cls_w.reshape(NCLS, OC).astype(jnp.bfloat16)
    bc2 = cls_b.reshape(NCLS, 1).astype(jnp.float32)
    ah = jnp.asarray(_bilinear_matrix(H, Ho), jnp.bfloat16)
    awt = _bilinear_matrix(W, Wo).T
    awt2 = np.zeros((2 * Wo, 2 * W), np.float32)
    awt2[:Wo, :W] = awt
    awt2[Wo:, W:] = awt
    awt2 = jnp.asarray(awt2, jnp.bfloat16)

    out = pl.pallas_call(
        _fused_kernel,
        out_shape=jax.ShapeDtypeStruct((N, NCLS, H, W), jnp.float32),
        grid=(N // 2,),
        in_specs=[
            pl.BlockSpec((1, Hp, K18, 2 * Wo), lambda n: (n, 0, 0, 0)),
            pl.BlockSpec((OC, K18), lambda n: (0, 0)),
            pl.BlockSpec((OC, K18), lambda n: (0, 0)),
            pl.BlockSpec((OC, 1), lambda n: (0, 0)),
            pl.BlockSpec((NCLS, OC), lambda n: (0, 0)),
            pl.BlockSpec((NCLS, 1), lambda n: (0, 0)),
            pl.BlockSpec((H, Ho), lambda n: (0, 0)),
            pl.BlockSpec((2 * Wo, 2 * W), lambda n: (0, 0)),
        ],
        out_specs=pl.BlockSpec((2, NCLS, H, W), lambda n: (n, 0, 0, 0)),
        compiler_params=pltpu.CompilerParams(dimension_semantics=("parallel",)),
    )(Z, wa, wb2, bb2, wc2, bc2, ah, awt2)
    return out


# speedup vs baseline: 13.5216x; 13.5216x over previous
import numpy as np
import jax
import jax.numpy as jnp
from jax.experimental import pallas as pl
from jax.experimental.pallas import tpu as pltpu


def _bilinear_matrix(out_size, in_size):
    scale = in_size / out_size
    idx = np.arange(out_size)
    src = (idx + 0.5) * scale - 0.5
    src = np.clip(src, 0.0, None)
    i0 = np.minimum(np.floor(src).astype(np.int64), in_size - 1)
    i1 = np.minimum(i0 + 1, in_size - 1)
    w1 = (src - i0).astype(np.float32)
    w0 = (1.0 - w1).astype(np.float32)
    A = np.zeros((out_size, in_size), dtype=np.float32)
    A[idx, i0] += w0
    A[idx, i1] += w1
    return A


def _fused_kernel(p_ref, w27_ref, bb_ref, wc_ref, bc_ref, ah_ref, awt2_ref,
                  out_ref):
    _, NCLS, H, W = out_ref.shape
    K27, Ho, Wo = p_ref.shape[1:]
    OC = w27_ref.shape[0]

    P = jnp.concatenate([p_ref[0], p_ref[1]], axis=2)

    feat = jax.lax.dot_general(
        w27_ref[...], P, (((1,), (0,)), ((), ())),
        preferred_element_type=jnp.float32)
    feat = jnp.maximum(feat + bb_ref[...][:, :, None], 0.0).astype(jnp.bfloat16)

    logits = jax.lax.dot_general(
        wc_ref[...], feat, (((1,), (0,)), ((), ())),
        preferred_element_type=jnp.float32)
    logits = (logits + bc_ref[...][:, :, None]).astype(jnp.bfloat16)

    awt_b = jnp.broadcast_to(awt2_ref[...], (NCLS, 2 * Wo, 2 * W))
    t = jax.lax.dot_general(
        logits, awt_b, (((2,), (1,)), ((0,), (0,))),
        preferred_element_type=jnp.float32)
    t = t.astype(jnp.bfloat16)

    ah_b = jnp.broadcast_to(ah_ref[...], (NCLS, H, Ho))
    y = jax.lax.dot_general(
        ah_b, t, (((2,), (1,)), ((0,), (0,))),
        preferred_element_type=jnp.float32)
    out_ref[0] = y[:, :, :W]
    out_ref[1] = y[:, :, W:]


def kernel(x, backbone_w, backbone_b, cls_w, cls_b):
    N, C, H, W = x.shape
    OC = backbone_w.shape[0]
    NCLS = cls_w.shape[0]
    Ho = (H + 2 - 3) // 2 + 1
    Wo = (W + 2 - 3) // 2 + 1
    Hp, Wp = Ho + 1, Wo + 1

    xb = x.astype(jnp.bfloat16)
    xpad = jnp.pad(xb, ((0, 0), (0, 0), (1, 2 * Hp - H - 1),
                        (1, 2 * Wp - W - 1)))
    ph = xpad.reshape(N, C, Hp, 2, Wp, 2)
    ph = ph.transpose(0, 3, 5, 1, 2, 4).reshape(N, 4 * C, Hp, Wp)

    taps = []
    for i in range(3):
        for j in range(3):
            q = (2 * (i % 2) + (j % 2)) * C
            a, b = i // 2, j // 2
            taps.append(ph[:, q:q + C, a:a + Ho, b:b + Wo])
    P = jnp.stack(taps, axis=1)
    P = P.reshape(N, 9 * C, Ho, Wo)

    w27 = backbone_w.transpose(0, 2, 3, 1).reshape(OC, 9 * C)
    w27 = w27.astype(jnp.bfloat16)
    bb2 = backbone_b.reshape(OC, 1).astype(jnp.float32)
    wc2 = cls_w.reshape(NCLS, OC).astype(jnp.bfloat16)
    bc2 = cls_b.reshape(NCLS, 1).astype(jnp.float32)
    ah = jnp.asarray(_bilinear_matrix(H, Ho), jnp.bfloat16)
    awt = _bilinear_matrix(W, Wo).T
    awt2 = np.zeros((2 * Wo, 2 * W), np.float32)
    awt2[:Wo, :W] = awt
    awt2[Wo:, W:] = awt
    awt2 = jnp.asarray(awt2, jnp.bfloat16)

    out = pl.pallas_call(
        _fused_kernel,
        out_shape=jax.ShapeDtypeStruct((N, NCLS, H, W), jnp.float32),
        grid=(N // 2,),
        in_specs=[
            pl.BlockSpec((2, 9 * C, Ho, Wo), lambda n: (n, 0, 0, 0)),
            pl.BlockSpec((OC, 9 * C), lambda n: (0, 0)),
            pl.BlockSpec((OC, 1), lambda n: (0, 0)),
            pl.BlockSpec((NCLS, OC), lambda n: (0, 0)),
            pl.BlockSpec((NCLS, 1), lambda n: (0, 0)),
            pl.BlockSpec((H, Ho), lambda n: (0, 0)),
            pl.BlockSpec((2 * Wo, 2 * W), lambda n: (0, 0)),
        ],
        out_specs=pl.BlockSpec((2, NCLS, H, W), lambda n: (n, 0, 0, 0)),
        compiler_params=pltpu.CompilerParams(dimension_semantics=("parallel",)),
    )(P, w27, bb2, wc2, bc2, ah, awt2)
    return out


# speedup vs baseline: 26.3779x; 1.9508x over previous
import numpy as np
import jax
import jax.numpy as jnp
from jax.experimental import pallas as pl
from jax.experimental.pallas import tpu as pltpu


def _bilinear_matrix(out_size, in_size):
    scale = in_size / out_size
    idx = np.arange(out_size)
    src = (idx + 0.5) * scale - 0.5
    src = np.clip(src, 0.0, None)
    i0 = np.minimum(np.floor(src).astype(np.int64), in_size - 1)
    i1 = np.minimum(i0 + 1, in_size - 1)
    w1 = (src - i0).astype(np.float32)
    w0 = (1.0 - w1).astype(np.float32)
    A = np.zeros((out_size, in_size), dtype=np.float32)
    A[idx, i0] += w0
    A[idx, i1] += w1
    return A


def _fused_kernel(x_ref, s_ref, r_ref, w27_ref, bb_ref, wc_ref, bc_ref,
                  ah_ref, awt2_ref, out_ref):
    _, NCLS, H, W = out_ref.shape
    B, C, _, _ = x_ref.shape
    Ho = ah_ref.shape[1]
    Wo = s_ref.shape[1] // 3
    OC = w27_ref.shape[0]

    xb = x_ref[...].astype(jnp.bfloat16).reshape(B * C * H, W)

    U = jax.lax.dot_general(
        xb, s_ref[...], (((1,), (0,)), ((), ())),
        preferred_element_type=jnp.float32).astype(jnp.bfloat16)
    U = U.reshape(B * C, H, 3 * Wo)

    r_b = jnp.broadcast_to(r_ref[...], (B * C, 3 * Ho, H))
    rj = []
    for j in range(3):
        rj.append(jax.lax.dot_general(
            r_b, U[:, :, j * Wo:(j + 1) * Wo], (((2,), (1,)), ((0,), (0,))),
            preferred_element_type=jnp.float32).astype(jnp.bfloat16))

    slabs = []
    for i in range(3):
        for j in range(3):
            s0 = rj[j][0:C, i * Ho:(i + 1) * Ho, :]
            s1 = rj[j][C:2 * C, i * Ho:(i + 1) * Ho, :]
            slabs.append(jnp.concatenate([s0, s1], axis=2))
    P = jnp.concatenate(slabs, axis=0)

    feat = jax.lax.dot_general(
        w27_ref[...], P, (((1,), (0,)), ((), ())),
        preferred_element_type=jnp.float32)
    feat = jnp.maximum(feat + bb_ref[...][:, :, None], 0.0).astype(jnp.bfloat16)

    logits = jax.lax.dot_general(
        wc_ref[...], feat, (((1,), (0,)), ((), ())),
        preferred_element_type=jnp.float32)
    logits = (logits + bc_ref[...][:, :, None]).astype(jnp.bfloat16)

    awt_b = jnp.broadcast_to(awt2_ref[...], (NCLS, 2 * Wo, 2 * W))
    t = jax.lax.dot_general(
        logits, awt_b, (((2,), (1,)), ((0,), (0,))),
        preferred_element_type=jnp.float32)
    t = t.astype(jnp.bfloat16)

    ah_b = jnp.broadcast_to(ah_ref[...], (NCLS, H, Ho))
    y = jax.lax.dot_general(
        ah_b, t, (((2,), (1,)), ((0,), (0,))),
        preferred_element_type=jnp.float32)
    out_ref[0] = y[:, :, :W]
    out_ref[1] = y[:, :, W:]


def kernel(x, backbone_w, backbone_b, cls_w, cls_b):
    N, C, H, W = x.shape
    OC = backbone_w.shape[0]
    NCLS = cls_w.shape[0]
    Ho = (H + 2 - 3) // 2 + 1
    Wo = (W + 2 - 3) // 2 + 1

    S = np.zeros((W, 3 * Wo), np.float32)
    for j in range(3):
        for wo in range(Wo):
            w = 2 * wo + j - 1
            if 0 <= w < W:
                S[w, j * Wo + wo] = 1.0
    R = np.zeros((3 * Ho, H), np.float32)
    for i in range(3):
        for ho in range(Ho):
            h = 2 * ho + i - 1
            if 0 <= h < H:
                R[i * Ho + ho, h] = 1.0
    S = jnp.asarray(S, jnp.bfloat16)
    R = jnp.asarray(R, jnp.bfloat16)

    w27 = backbone_w.transpose(0, 2, 3, 1).reshape(OC, 9 * C)
    w27 = w27.astype(jnp.bfloat16)
    bb2 = backbone_b.reshape(OC, 1).astype(jnp.float32)
    wc2 = cls_w.reshape(NCLS, OC).astype(jnp.bfloat16)
    bc2 = cls_b.reshape(NCLS, 1).astype(jnp.float32)
    ah = jnp.asarray(_bilinear_matrix(H, Ho), jnp.bfloat16)
    awt = _bilinear_matrix(W, Wo).T
    awt2 = np.zeros((2 * Wo, 2 * W), np.float32)
    awt2[:Wo, :W] = awt
    awt2[Wo:, W:] = awt
    awt2 = jnp.asarray(awt2, jnp.bfloat16)

    out = pl.pallas_call(
        _fused_kernel,
        out_shape=jax.ShapeDtypeStruct((N, NCLS, H, W), jnp.float32),
        grid=(N // 2,),
        in_specs=[
            pl.BlockSpec((2, C, H, W), lambda n: (n, 0, 0, 0)),
            pl.BlockSpec((W, 3 * Wo), lambda n: (0, 0)),
            pl.BlockSpec((3 * Ho, H), lambda n: (0, 0)),
            pl.BlockSpec((OC, 9 * C), lambda n: (0, 0)),
            pl.BlockSpec((OC, 1), lambda n: (0, 0)),
            pl.BlockSpec((NCLS, OC), lambda n: (0, 0)),
            pl.BlockSpec((NCLS, 1), lambda n: (0, 0)),
            pl.BlockSpec((H, Ho), lambda n: (0, 0)),
            pl.BlockSpec((2 * Wo, 2 * W), lambda n: (0, 0)),
        ],
        out_specs=pl.BlockSpec((2, NCLS, H, W), lambda n: (n, 0, 0, 0)),
        compiler_params=pltpu.CompilerParams(dimension_semantics=("parallel",)),
    )(x, S, R, w27, bb2, wc2, bc2, ah, awt2)
    return out


# speedup vs baseline: 32.4908x; 1.2317x over previous
import numpy as np
import jax
import jax.numpy as jnp
from jax.experimental import pallas as pl
from jax.experimental.pallas import tpu as pltpu

_PK = 4


def _bilinear_matrix(out_size, in_size):
    scale = in_size / out_size
    idx = np.arange(out_size)
    src = (idx + 0.5) * scale - 0.5
    src = np.clip(src, 0.0, None)
    i0 = np.minimum(np.floor(src).astype(np.int64), in_size - 1)
    i1 = np.minimum(i0 + 1, in_size - 1)
    w1 = (src - i0).astype(np.float32)
    w0 = (1.0 - w1).astype(np.float32)
    A = np.zeros((out_size, in_size), dtype=np.float32)
    A[idx, i0] += w0
    A[idx, i1] += w1
    return A


def _fused_kernel(x0_ref, x1_ref, x2_ref, x3_ref, s_ref, r_ref, w27_ref,
                  bb_ref, wc_ref, bc_ref, ah_ref, awt_ref, out_ref):
    _, NCLS, H, W = out_ref.shape
    C = x0_ref.shape[1]
    Ho = ah_ref.shape[1]
    WL = _PK * W
    Wo = W // 2
    WoL = _PK * Wo
    OC = w27_ref.shape[0]

    xcat = jnp.concatenate(
        [x0_ref[0], x1_ref[0], x2_ref[0], x3_ref[0]], axis=2)
    xf = xcat.astype(jnp.bfloat16).reshape(C * H, WL)

    U = jax.lax.dot_general(
        xf, s_ref[...], (((1,), (0,)), ((), ())),
        preferred_element_type=jnp.float32).astype(jnp.bfloat16)
    U = U.reshape(C, H, 3 * WoL)

    r_b = jnp.broadcast_to(r_ref[...], (C, 3 * Ho, H))
    rj = []
    for j in range(3):
        rj.append(jax.lax.dot_general(
            r_b, U[:, :, j * WoL:(j + 1) * WoL], (((2,), (1,)), ((0,), (0,))),
            preferred_element_type=jnp.float32).astype(jnp.bfloat16))

    slabs = []
    for i in range(3):
        for j in range(3):
            slabs.append(rj[j][:, i * Ho:(i + 1) * Ho, :])
    P = jnp.concatenate(slabs, axis=0)

    feat = jax.lax.dot_general(
        w27_ref[...], P, (((1,), (0,)), ((), ())),
        preferred_element_type=jnp.float32)
    feat = jnp.maximum(feat + bb_ref[...][:, :, None], 0.0).astype(jnp.bfloat16)

    logits = jax.lax.dot_general(
        wc_ref[...], feat, (((1,), (0,)), ((), ())),
        preferred_element_type=jnp.float32)
    logits = (logits + bc_ref[...][:, :, None]).astype(jnp.bfloat16)

    t = jax.lax.dot_general(
        logits, awt_ref[...], (((2,), (0,)), ((), ())),
        preferred_element_type=jnp.float32)
    t = t.astype(jnp.bfloat16)

    ah_b = jnp.broadcast_to(ah_ref[...], (NCLS, H, Ho))
    y = jax.lax.dot_general(
        ah_b, t, (((2,), (1,)), ((0,), (0,))),
        preferred_element_type=jnp.float32)
    for k in range(_PK):
        out_ref[k] = y[:, :, k * W:(k + 1) * W]


def kernel(x, backbone_w, backbone_b, cls_w, cls_b):
    N, C, H, W = x.shape
    OC = backbone_w.shape[0]
    NCLS = cls_w.shape[0]
    Ho = (H + 2 - 3) // 2 + 1
    Wo = (W + 2 - 3) // 2 + 1
    WoL = _PK * Wo

    S = np.zeros((_PK * W, 3 * WoL), np.float32)
    for img in range(_PK):
        for j in range(3):
            for wo in range(Wo):
                w = 2 * wo + j - 1
                if 0 <= w < W:
                    S[img * W + w, j * WoL + img * Wo + wo] = 1.0
    R = np.zeros((3 * Ho, H), np.float32)
    for i in range(3):
        for ho in range(Ho):
            h = 2 * ho + i - 1
            if 0 <= h < H:
                R[i * Ho + ho, h] = 1.0
    S = jnp.asarray(S, jnp.bfloat16)
    R = jnp.asarray(R, jnp.bfloat16)

    w27 = backbone_w.transpose(0, 2, 3, 1).reshape(OC, 9 * C)
    w27 = w27.astype(jnp.bfloat16)
    bb2 = backbone_b.reshape(OC, 1).astype(jnp.float32)
    wc2 = cls_w.reshape(NCLS, OC).astype(jnp.bfloat16)
    bc2 = cls_b.reshape(NCLS, 1).astype(jnp.float32)
    ah = jnp.asarray(_bilinear_matrix(H, Ho), jnp.bfloat16)
    awt = _bilinear_matrix(W, Wo).T
    awt4 = np.zeros((WoL, _PK * W), np.float32)
    for img in range(_PK):
        awt4[img * Wo:(img + 1) * Wo, img * W:(img + 1) * W] = awt
    awt4 = jnp.asarray(awt4, jnp.bfloat16)

    x_specs = [
        pl.BlockSpec((1, C, H, W), lambda n, k=k: (_PK * n + k, 0, 0, 0))
        for k in range(_PK)
    ]
    out = pl.pallas_call(
        _fused_kernel,
        out_shape=jax.ShapeDtypeStruct((N, NCLS, H, W), jnp.float32),
        grid=(N // _PK,),
        in_specs=x_specs + [
            pl.BlockSpec((_PK * W, 3 * WoL), lambda n: (0, 0)),
            pl.BlockSpec((3 * Ho, H), lambda n: (0, 0)),
            pl.BlockSpec((OC, 9 * C), lambda n: (0, 0)),
            pl.BlockSpec((OC, 1), lambda n: (0, 0)),
            pl.BlockSpec((NCLS, OC), lambda n: (0, 0)),
            pl.BlockSpec((NCLS, 1), lambda n: (0, 0)),
            pl.BlockSpec((H, Ho), lambda n: (0, 0)),
            pl.BlockSpec((WoL, _PK * W), lambda n: (0, 0)),
        ],
        out_specs=pl.BlockSpec((_PK, NCLS, H, W), lambda n: (n, 0, 0, 0)),
        compiler_params=pltpu.CompilerParams(dimension_semantics=("parallel",)),
    )(x, x, x, x, S, R, w27, bb2, wc2, bc2, ah, awt4)
    return out
